# Initial kernel scaffold; baseline (speedup 1.0000x reference)
import jax
import jax.numpy as jnp
from jax.experimental import pallas as pl


def kernel(hidden_states, weight):
    raise NotImplementedError("write your pallas kernel here")


# speedup vs baseline: 1.7550x; 1.7550x over previous
import functools

import jax
import jax.numpy as jnp
from jax import lax
from jax.experimental import pallas as pl
from jax.experimental.pallas import tpu as pltpu

_E = 64
_TOPK = 8
_ALPHA = 0.01


def _router_body(x_ref, w_ref, idx_ref, wgt_ref, aux_ref, cnt_acc, ssum_acc,
                 *, bt, seq_len, bsz, nblocks):
    i = pl.program_id(0)
    x = x_ref[...]
    w = w_ref[...]

    logits = lax.dot_general(
        x, w, dimension_numbers=(((1,), (1,)), ((), ())),
        preferred_element_type=jnp.float32)

    m = jnp.max(logits, axis=1, keepdims=True)
    e = jnp.exp(logits - m)
    scores = e / jnp.sum(e, axis=1, keepdims=True)

    ids = lax.broadcasted_iota(jnp.int32, (bt, _E), 1)
    work = scores
    vals = []
    idxs = []
    for _ in range(_TOPK):
        cur = jnp.max(work, axis=1, keepdims=True)
        sel_idx = jnp.min(jnp.where(work == cur, ids, _E), axis=1,
                          keepdims=True)
        vals.append(cur)
        idxs.append(sel_idx)
        work = jnp.where(ids == sel_idx, -jnp.inf, work)

    topk_w = jnp.concatenate(vals, axis=1)
    topk_i = jnp.concatenate(idxs, axis=1)
    denom = jnp.sum(topk_w, axis=1, keepdims=True) + 1e-20
    idx_ref[...] = topk_i
    wgt_ref[...] = topk_w / denom

    sel_mask = (work == -jnp.inf).astype(jnp.float32)
    cnt_e = jnp.sum(sel_mask, axis=0, keepdims=True)
    ssum_e = jnp.sum(scores, axis=0, keepdims=True)

    blocks_per_batch = seq_len // bt
    b = i // blocks_per_batch
    onehot_b = (lax.broadcasted_iota(jnp.int32, (bsz, 1), 0) == b
                ).astype(jnp.float32)

    @pl.when(i == 0)
    def _init():
        cnt_acc[...] = onehot_b * cnt_e
        ssum_acc[...] = onehot_b * ssum_e

    @pl.when(i > 0)
    def _accum():
        cnt_acc[...] += onehot_b * cnt_e
        ssum_acc[...] += onehot_b * ssum_e

    @pl.when(i == nblocks - 1)
    def _finalize():
        scale = _ALPHA * (_E / (seq_len * _TOPK)) / (seq_len * bsz)
        aux_ref[...] = (jnp.sum(cnt_acc[...] * ssum_acc[...]) * scale
                        ).reshape(1, 1)


@functools.partial(jax.jit, static_argnames=("bt",))
def _router(hidden_flat, weight, bt):
    n, d = hidden_flat.shape
    bsz = 4
    seq_len = n // bsz
    nblocks = n // bt

    body = functools.partial(_router_body, bt=bt, seq_len=seq_len, bsz=bsz,
                             nblocks=nblocks)
    topk_i, topk_w, aux = pl.pallas_call(
        body,
        grid=(nblocks,),
        in_specs=[
            pl.BlockSpec((bt, d), lambda i: (i, 0)),
            pl.BlockSpec((_E, d), lambda i: (0, 0)),
        ],
        out_specs=[
            pl.BlockSpec((bt, _TOPK), lambda i: (i, 0)),
            pl.BlockSpec((bt, _TOPK), lambda i: (i, 0)),
            pl.BlockSpec((1, 1), lambda i: (0, 0)),
        ],
        out_shape=[
            jax.ShapeDtypeStruct((n, _TOPK), jnp.int32),
            jax.ShapeDtypeStruct((n, _TOPK), jnp.float32),
            jax.ShapeDtypeStruct((1, 1), jnp.float32),
        ],
        scratch_shapes=[
            pltpu.VMEM((bsz, _E), jnp.float32),
            pltpu.VMEM((bsz, _E), jnp.float32),
        ],
        compiler_params=pltpu.CompilerParams(
            dimension_semantics=("arbitrary",)),
    )(hidden_flat, weight)
    return topk_i, topk_w, aux[0, 0]


def kernel(hidden_states, weight):
    bsz, seq_len, d = hidden_states.shape
    hidden_flat = hidden_states.reshape(-1, d)
    return _router(hidden_flat, weight, bt=1024)


# speedup vs baseline: 3.5217x; 2.0066x over previous
import functools

import jax
import jax.numpy as jnp
from jax import lax
from jax.experimental import pallas as pl
from jax.experimental.pallas import tpu as pltpu

_E = 64
_TOPK = 8
_ALPHA = 0.01


def _router_body(x_ref, w_ref, idx_ref, wgt_ref, aux_ref, cnt_acc, ssum_acc,
                 *, bt, seq_len, bsz, nblocks):
    i = pl.program_id(0)
    x = x_ref[...]
    w = w_ref[...]

    logits = lax.dot_general(
        w, x, dimension_numbers=(((1,), (1,)), ((), ())),
        preferred_element_type=jnp.float32)

    m = jnp.max(logits, axis=0, keepdims=True)
    e = jnp.exp(logits - m)
    scores = e / jnp.sum(e, axis=0, keepdims=True)

    ids = lax.broadcasted_iota(jnp.int32, (_E, bt), 0)
    work = scores
    vals = []
    idxs = []
    for _ in range(_TOPK):
        cur = jnp.max(work, axis=0, keepdims=True)
        sel_idx = jnp.min(jnp.where(work == cur, ids, _E), axis=0,
                          keepdims=True)
        vals.append(cur)
        idxs.append(sel_idx)
        work = jnp.where(ids == sel_idx, -jnp.inf, work)

    topk_w = jnp.concatenate(vals, axis=0)
    topk_i = jnp.concatenate(idxs, axis=0)
    denom = jnp.sum(topk_w, axis=0, keepdims=True) + 1e-20
    idx_ref[...] = topk_i
    wgt_ref[...] = topk_w / denom

    sel_mask = (work == -jnp.inf).astype(jnp.float32)
    cnt_e = jnp.sum(sel_mask, axis=1, keepdims=True)
    ssum_e = jnp.sum(scores, axis=1, keepdims=True)

    blocks_per_batch = seq_len // bt
    b = i // blocks_per_batch
    onehot_b = (lax.broadcasted_iota(jnp.int32, (1, bsz), 1) == b
                ).astype(jnp.float32)

    @pl.when(i == 0)
    def _init():
        cnt_acc[...] = cnt_e * onehot_b
        ssum_acc[...] = ssum_e * onehot_b

    @pl.when(i > 0)
    def _accum():
        cnt_acc[...] += cnt_e * onehot_b
        ssum_acc[...] += ssum_e * onehot_b

    @pl.when(i == nblocks - 1)
    def _finalize():
        scale = _ALPHA * (_E / (seq_len * _TOPK)) / (seq_len * bsz)
        aux_ref[...] = (jnp.sum(cnt_acc[...] * ssum_acc[...]) * scale
                        ).reshape(1, 1)


@functools.partial(jax.jit, static_argnames=("bt",))
def _router(hidden_flat, weight, bt):
    n, d = hidden_flat.shape
    bsz = 4
    seq_len = n // bsz
    nblocks = n // bt

    body = functools.partial(_router_body, bt=bt, seq_len=seq_len, bsz=bsz,
                             nblocks=nblocks)
    topk_i, topk_w, aux = pl.pallas_call(
        body,
        grid=(nblocks,),
        in_specs=[
            pl.BlockSpec((bt, d), lambda i: (i, 0)),
            pl.BlockSpec((_E, d), lambda i: (0, 0)),
        ],
        out_specs=[
            pl.BlockSpec((_TOPK, bt), lambda i: (0, i)),
            pl.BlockSpec((_TOPK, bt), lambda i: (0, i)),
            pl.BlockSpec((1, 1), lambda i: (0, 0)),
        ],
        out_shape=[
            jax.ShapeDtypeStruct((_TOPK, n), jnp.int32),
            jax.ShapeDtypeStruct((_TOPK, n), jnp.float32),
            jax.ShapeDtypeStruct((1, 1), jnp.float32),
        ],
        scratch_shapes=[
            pltpu.VMEM((_E, bsz), jnp.float32),
            pltpu.VMEM((_E, bsz), jnp.float32),
        ],
        compiler_params=pltpu.CompilerParams(
            dimension_semantics=("arbitrary",)),
    )(hidden_flat, weight)
    return topk_i, topk_w, aux[0, 0]


def kernel(hidden_states, weight):
    bsz, seq_len, d = hidden_states.shape
    hidden_flat = hidden_states.reshape(-1, d)
    topk_i_t, topk_w_t, aux = _router(hidden_flat, weight, bt=1024)
    return topk_i_t.T, topk_w_t.T, aux


# speedup vs baseline: 3.7235x; 1.0573x over previous
import functools

import jax
import jax.numpy as jnp
from jax import lax
from jax.experimental import pallas as pl
from jax.experimental.pallas import tpu as pltpu

_E = 64
_TOPK = 8
_ALPHA = 0.01


def _router_body(x_ref, w_ref, idx_ref, wgt_ref, aux_ref, cnt_acc, ssum_acc,
                 *, bt, seq_len, bsz, nblocks):
    i = pl.program_id(0)
    x = x_ref[...]
    w = w_ref[...]

    logits = lax.dot_general(
        w, x, dimension_numbers=(((1,), (1,)), ((), ())),
        preferred_element_type=jnp.float32)

    m = jnp.max(logits, axis=0, keepdims=True)
    e = jnp.exp(logits - m)
    scores = e / jnp.sum(e, axis=0, keepdims=True)

    ids = lax.broadcasted_iota(jnp.int32, (_E, bt), 0)
    work = scores
    vals = []
    idxs = []
    for _ in range(_TOPK):
        cur = jnp.max(work, axis=0, keepdims=True)
        sel_idx = jnp.min(jnp.where(work == cur, ids, _E), axis=0,
                          keepdims=True)
        vals.append(cur)
        idxs.append(sel_idx)
        work = jnp.where(ids == sel_idx, -jnp.inf, work)

    topk_w = jnp.concatenate(vals, axis=0)
    topk_i = jnp.concatenate(idxs, axis=0)
    denom = jnp.sum(topk_w, axis=0, keepdims=True) + 1e-20
    idx_ref[...] = topk_i
    wgt_ref[...] = topk_w / denom

    sel_mask = (work == -jnp.inf).astype(jnp.float32)
    cnt_e = jnp.sum(sel_mask, axis=1, keepdims=True)
    ssum_e = jnp.sum(scores, axis=1, keepdims=True)

    blocks_per_batch = seq_len // bt
    b = i // blocks_per_batch
    onehot_b = (lax.broadcasted_iota(jnp.int32, (1, bsz), 1) == b
                ).astype(jnp.float32)

    @pl.when(i == 0)
    def _init():
        cnt_acc[...] = cnt_e * onehot_b
        ssum_acc[...] = ssum_e * onehot_b

    @pl.when(i > 0)
    def _accum():
        cnt_acc[...] += cnt_e * onehot_b
        ssum_acc[...] += ssum_e * onehot_b

    @pl.when(i == nblocks - 1)
    def _finalize():
        scale = _ALPHA * (_E / (seq_len * _TOPK)) / (seq_len * bsz)
        aux_ref[...] = (jnp.sum(cnt_acc[...] * ssum_acc[...]) * scale
                        ).reshape(1, 1)


@functools.partial(jax.jit, static_argnames=("bt",))
def _router(hidden_flat, weight, bt):
    n, d = hidden_flat.shape
    bsz = 4
    seq_len = n // bsz
    nblocks = n // bt

    body = functools.partial(_router_body, bt=bt, seq_len=seq_len, bsz=bsz,
                             nblocks=nblocks)
    topk_i, topk_w, aux = pl.pallas_call(
        body,
        grid=(nblocks,),
        in_specs=[
            pl.BlockSpec((bt, d), lambda i: (i, 0)),
            pl.BlockSpec((_E, d), lambda i: (0, 0)),
        ],
        out_specs=[
            pl.BlockSpec((_TOPK, bt), lambda i: (0, i)),
            pl.BlockSpec((_TOPK, bt), lambda i: (0, i)),
            pl.BlockSpec((1, 1), lambda i: (0, 0)),
        ],
        out_shape=[
            jax.ShapeDtypeStruct((_TOPK, n), jnp.int32),
            jax.ShapeDtypeStruct((_TOPK, n), jnp.float32),
            jax.ShapeDtypeStruct((1, 1), jnp.float32),
        ],
        scratch_shapes=[
            pltpu.VMEM((_E, bsz), jnp.float32),
            pltpu.VMEM((_E, bsz), jnp.float32),
        ],
        compiler_params=pltpu.CompilerParams(
            dimension_semantics=("arbitrary",)),
    )(hidden_flat, weight)
    return topk_i, topk_w, aux[0, 0]


def kernel(hidden_states, weight):
    bsz, seq_len, d = hidden_states.shape
    hidden_flat = hidden_states.reshape(-1, d)
    topk_i_t, topk_w_t, aux = _router(hidden_flat, weight, bt=2048)
    return topk_i_t.T, topk_w_t.T, aux
